# baseline (speedup 1.0000x reference)
import jax
import jax.numpy as jnp
from jax.experimental import pallas as pl


def _copy_kernel(x_ref, o_ref):
    o_ref[...] = x_ref[...]


def kernel(x, W, b):
    n, f = x.shape
    blk = 2000
    return pl.pallas_call(
        _copy_kernel,
        grid=(n // blk,),
        in_specs=[pl.BlockSpec((blk, f), lambda i: (i, 0))],
        out_specs=pl.BlockSpec((blk, f), lambda i: (i, 0)),
        out_shape=jax.ShapeDtypeStruct((n, f), x.dtype),
    )(x)


# speedup vs baseline: 1.7471x; 1.7471x over previous
import jax
import jax.numpy as jnp
from jax.experimental import pallas as pl
from jax.experimental.pallas import tpu as pltpu

_CHUNKS = 10


def _slice_copy_kernel(x_hbm, o_hbm, vin, vout, in_sems, out_sems):
    n, f = vin.shape
    r = n // _CHUNKS
    for c in range(_CHUNKS):
        pltpu.make_async_copy(
            x_hbm.at[pl.ds(c * r, r), :], vin.at[pl.ds(c * r, r), :], in_sems.at[c]
        ).start()
    for c in range(_CHUNKS):
        pltpu.make_async_copy(
            x_hbm.at[pl.ds(c * r, r), :], vin.at[pl.ds(c * r, r), :], in_sems.at[c]
        ).wait()
        vout[pl.ds(c * r, r), :] = vin[pl.ds(c * r, r), 3:]
        pltpu.make_async_copy(
            vout.at[pl.ds(c * r, r), :], o_hbm.at[pl.ds(c * r, r), :], out_sems.at[c]
        ).start()
    for c in range(_CHUNKS):
        pltpu.make_async_copy(
            vout.at[pl.ds(c * r, r), :], o_hbm.at[pl.ds(c * r, r), :], out_sems.at[c]
        ).wait()


def kernel(x, W, b):
    n, f = x.shape
    fo = f - 3
    return pl.pallas_call(
        _slice_copy_kernel,
        in_specs=[pl.BlockSpec(memory_space=pltpu.MemorySpace.HBM)],
        out_specs=pl.BlockSpec(memory_space=pltpu.MemorySpace.HBM),
        out_shape=jax.ShapeDtypeStruct((n, fo), x.dtype),
        scratch_shapes=[
            pltpu.VMEM((n, f), x.dtype),
            pltpu.VMEM((n, fo), x.dtype),
            pltpu.SemaphoreType.DMA((_CHUNKS,)),
            pltpu.SemaphoreType.DMA((_CHUNKS,)),
        ],
    )(x)


# speedup vs baseline: 3.6276x; 2.0763x over previous
import jax
from jax.experimental import pallas as pl


def _slice_transpose_kernel(xt_ref, o_ref):
    o_ref[...] = xt_ref[3:, :].T


def kernel(x, W, b):
    n, f = x.shape
    fo = f - 3
    xt = x.T
    blkc = 2048
    return pl.pallas_call(
        _slice_transpose_kernel,
        grid=(pl.cdiv(n, blkc),),
        in_specs=[pl.BlockSpec((f, blkc), lambda i: (0, i))],
        out_specs=pl.BlockSpec((blkc, fo), lambda i: (i, 0)),
        out_shape=jax.ShapeDtypeStruct((n, fo), x.dtype),
    )(xt)
